# baseline (speedup 1.0000x reference)
import functools

import numpy as np

import jax
import jax.numpy as jnp
from jax import lax
from jax.experimental import pallas as pl
from jax.experimental.pallas import tpu as pltpu
from jax.experimental.pallas import tpu_sc as plsc

N_PRED = 20000
N_GT = 20000
IOU_THRESH = 0.5

ROWS_T = 512
NR = 20480
COLS = 20480
CT = 2048
NCH = COLS // CT
SENT = N_GT
MATCH = 20480
NV = NR // 16
DTOT = float(np.float32(np.float32(float(N_GT)) + np.float32(1e-6)))


def _phase1_body(dx, dy, dx2b, dy2b, da, dsc,
                 gx, gy, gx2b, gy2b, ga, ssc,
                 biou_ref, bidx_ref, brank_ref):
    i = pl.program_id(0)
    dxv = dx[:, :]
    dyv = dy[:, :]
    dx2 = dx2b[:, :]
    dy2 = dy2b[:, :]
    dav = da[:, :]
    dscv = dsc[:, :]
    rowid = i * ROWS_T + lax.broadcasted_iota(jnp.int32, (ROWS_T, 1), 0)
    colid0 = lax.broadcasted_iota(jnp.int32, (ROWS_T, CT), 1)

    rmax = jnp.full((ROWS_T, 1), -1.0, jnp.float32)
    ridx = jnp.zeros((ROWS_T, 1), jnp.int32)
    crank = jnp.zeros((ROWS_T, 1), jnp.int32)

    for c in range(NCH):
        sl = slice(c * CT, (c + 1) * CT)
        x1 = jnp.maximum(dxv, gx[:, sl])
        y1 = jnp.maximum(dyv, gy[:, sl])
        x2 = jnp.minimum(dx2, gx2b[:, sl])
        y2 = jnp.minimum(dy2, gy2b[:, sl])
        inter = jnp.maximum(x2 - x1, 0.0) * jnp.maximum(y2 - y1, 0.0)
        iou = inter / (((dav + ga[:, sl]) - inter) + 1e-6)
        cmax = jnp.max(iou, axis=1, keepdims=True)
        cidx = jnp.min(jnp.where(iou == cmax, colid0, CT),
                       axis=1, keepdims=True) + c * CT
        upd = cmax > rmax
        ridx = jnp.where(upd, cidx, ridx)
        rmax = jnp.where(upd, cmax, rmax)

        sv = ssc[:, sl]
        rmask = (sv > dscv) | ((sv == dscv) & (colid0 < (rowid - c * CT)))
        crank = crank + jnp.sum(rmask.astype(jnp.int32),
                                axis=1, keepdims=True)

    biou_ref[:, :] = rmax
    bidx_ref[:, :] = ridx
    brank_ref[:, :] = crank


_phase1 = pl.pallas_call(
    _phase1_body,
    grid=(NR // ROWS_T,),
    in_specs=(
        [pl.BlockSpec((ROWS_T, 1), lambda i: (i, 0)) for _ in range(6)]
        + [pl.BlockSpec((1, COLS), lambda i: (0, 0)) for _ in range(6)]
    ),
    out_specs=[pl.BlockSpec((ROWS_T, 1), lambda i: (i, 0)) for _ in range(3)],
    out_shape=[
        jax.ShapeDtypeStruct((NR, 1), jnp.float32),
        jax.ShapeDtypeStruct((NR, 1), jnp.int32),
        jax.ShapeDtypeStruct((NR, 1), jnp.int32),
    ],
)


@functools.partial(
    pl.kernel,
    out_type=jax.ShapeDtypeStruct((16,), jnp.float32),
    mesh=plsc.VectorSubcoreMesh(core_axis_name="c", subcore_axis_name="s"),
    compiler_params=pltpu.CompilerParams(needs_layout_passes=False),
    scratch_types=[
        pltpu.VMEM((NR,), jnp.float32),
        pltpu.VMEM((NR,), jnp.int32),
        pltpu.VMEM((NR,), jnp.int32),
        pltpu.VMEM((NR,), jnp.int32),
        pltpu.VMEM((MATCH,), jnp.int32),
        pltpu.VMEM((16,), jnp.float32),
    ],
)
def _phase2(biou_hbm, bidx_hbm, rank_hbm, out_hbm,
            biou_v, bidx_v, rank_v, sortedg_v, matched_v, ap_v):
    @pl.when((lax.axis_index("c") == 0) & (lax.axis_index("s") == 0))
    def _():
        pltpu.sync_copy(biou_hbm, biou_v)
        pltpu.sync_copy(bidx_hbm, bidx_v)
        pltpu.sync_copy(rank_hbm, rank_v)

        zero16 = jnp.zeros((16,), jnp.int32)

        def zinit(c, _):
            matched_v[pl.ds(c * 16, 16)] = zero16
            return 0

        lax.fori_loop(0, MATCH // 16, zinit, 0)

        def loop_a(c, _):
            ds = pl.ds(c * 16, 16)
            gg = jnp.where(biou_v[ds] > IOU_THRESH, bidx_v[ds], SENT)
            plsc.store_scatter(sortedg_v, [rank_v[ds]], gg)
            return 0

        lax.fori_loop(0, NV, loop_a, 0)

        iota16 = lax.broadcasted_iota(jnp.int32, (16,), 0)
        ones_i = jnp.ones((16,), jnp.int32)

        def loop_b(c, carry):
            cum_carry, ap_acc = carry
            gg = sortedg_v[pl.ds(c * 16, 16)]
            m = plsc.load_gather(matched_v, [gg])
            _, lastm = plsc.scan_count(lax.rev(gg, (0,)))
            firsti = lax.rev(lastm.astype(jnp.int32), (0,))
            tpm = (gg < SENT) & (m == 0) & (firsti == 1)
            plsc.store_scatter(matched_v, [gg], ones_i, mask=tpm)
            tpf = jnp.where(tpm, 1.0, 0.0).astype(jnp.float32)
            cum = plsc.cumsum(tpf) + cum_carry
            new_carry = jnp.broadcast_to(jnp.max(cum), (16,))
            p = iota16 + c * 16
            pf = p.astype(jnp.float32)
            cum_prev = cum - tpf
            d = cum / DTOT - cum_prev / DTOT
            prec = cum / ((pf + 1.0) + 1e-6)
            prec_prev = jnp.where(p == 0, 1.0, cum_prev / (pf + 1e-6))
            ap_acc = ap_acc + d * (prec + prec_prev) * 0.5
            return (new_carry, ap_acc)

        _, ap_acc = lax.fori_loop(
            0, NV, loop_b,
            (jnp.zeros((16,), jnp.float32), jnp.zeros((16,), jnp.float32)))
        ap_v[...] = jnp.broadcast_to(jnp.sum(ap_acc), (16,))
        pltpu.sync_copy(ap_v, out_hbm)


@jax.jit
def kernel(pred_boxes, pred_scores, gt_boxes, pred_labels, gt_labels):
    del pred_labels, gt_labels
    prow = NR - N_PRED
    gcol = COLS - N_GT
    dx = jnp.pad(pred_boxes[:, 0], (0, prow))
    dy = jnp.pad(pred_boxes[:, 1], (0, prow))
    dx2 = dx + jnp.pad(pred_boxes[:, 2], (0, prow))
    dy2 = dy + jnp.pad(pred_boxes[:, 3], (0, prow))
    da = jnp.abs((dx2 - dx) * (dy2 - dy))
    dsc = jnp.pad(pred_scores, (0, prow), constant_values=-1.0)
    gx = jnp.pad(gt_boxes[:, 0], (0, gcol), constant_values=5.0)
    gy = jnp.pad(gt_boxes[:, 1], (0, gcol), constant_values=5.0)
    gx2 = gx + jnp.pad(gt_boxes[:, 2], (0, gcol))
    gy2 = gy + jnp.pad(gt_boxes[:, 3], (0, gcol))
    ga = jnp.abs((gx2 - gx) * (gy2 - gy))
    ssc = jnp.pad(pred_scores, (0, COLS - N_PRED),
                  constant_values=-1.0)

    biou, bidx, brank = _phase1(
        dx[:, None], dy[:, None], dx2[:, None], dy2[:, None],
        da[:, None], dsc[:, None],
        gx[None, :], gy[None, :], gx2[None, :], gy2[None, :],
        ga[None, :], ssc[None, :])
    out16 = _phase2(biou.reshape(NR), bidx.reshape(NR), brank.reshape(NR))
    return out16[0]


# speedup vs baseline: 1.1262x; 1.1262x over previous
import functools

import numpy as np

import jax
import jax.numpy as jnp
from jax import lax
from jax.experimental import pallas as pl
from jax.experimental.pallas import tpu as pltpu
from jax.experimental.pallas import tpu_sc as plsc

N_PRED = 20000
N_GT = 20000
IOU_THRESH = 0.5

ROWS_T = 256
NR = 20224
COLS = 20480
CT = 4096
NCH = COLS // CT
SENT = N_GT
MATCH = 20480
NV = NR // 16
DTOT = float(np.float32(np.float32(float(N_GT)) + np.float32(1e-6)))


def _phase1_body(dx, dy, dx2b, dy2b, da, dsc,
                 gx, gy, gx2b, gy2b, ga, ssc,
                 biou_ref, bidx_ref, brank_ref):
    i = pl.program_id(0)
    dxv = dx[:, :]
    dyv = dy[:, :]
    dx2 = dx2b[:, :]
    dy2 = dy2b[:, :]
    dav = da[:, :]
    dscv = dsc[:, :]
    rowid = i * ROWS_T + lax.broadcasted_iota(jnp.int32, (ROWS_T, 1), 0)
    colid0 = lax.broadcasted_iota(jnp.int32, (ROWS_T, CT), 1)

    rmax = jnp.full((ROWS_T, 1), -1.0, jnp.float32)
    ridx = jnp.zeros((ROWS_T, 1), jnp.int32)
    crank = jnp.zeros((ROWS_T, 1), jnp.int32)

    for c in range(NCH):
        sl = slice(c * CT, (c + 1) * CT)
        x1 = jnp.maximum(dxv, gx[:, sl])
        y1 = jnp.maximum(dyv, gy[:, sl])
        x2 = jnp.minimum(dx2, gx2b[:, sl])
        y2 = jnp.minimum(dy2, gy2b[:, sl])
        inter = jnp.maximum(x2 - x1, 0.0) * jnp.maximum(y2 - y1, 0.0)
        iou = inter / (((dav + ga[:, sl]) - inter) + 1e-6)
        cmax = jnp.max(iou, axis=1, keepdims=True)
        cidx = jnp.min(jnp.where(iou == cmax, colid0, CT),
                       axis=1, keepdims=True) + c * CT
        upd = cmax > rmax
        ridx = jnp.where(upd, cidx, ridx)
        rmax = jnp.where(upd, cmax, rmax)

        sv = ssc[:, sl]
        rmask = (sv > dscv) | ((sv == dscv) & (colid0 < (rowid - c * CT)))
        crank = crank + jnp.sum(rmask.astype(jnp.int32),
                                axis=1, keepdims=True)

    biou_ref[:, :] = rmax
    bidx_ref[:, :] = ridx
    brank_ref[:, :] = crank


_phase1 = pl.pallas_call(
    _phase1_body,
    grid=(NR // ROWS_T,),
    in_specs=(
        [pl.BlockSpec((ROWS_T, 1), lambda i: (i, 0)) for _ in range(6)]
        + [pl.BlockSpec((1, COLS), lambda i: (0, 0)) for _ in range(6)]
    ),
    out_specs=[pl.BlockSpec((ROWS_T, 1), lambda i: (i, 0)) for _ in range(3)],
    out_shape=[
        jax.ShapeDtypeStruct((NR, 1), jnp.float32),
        jax.ShapeDtypeStruct((NR, 1), jnp.int32),
        jax.ShapeDtypeStruct((NR, 1), jnp.int32),
    ],
)


@functools.partial(
    pl.kernel,
    out_type=jax.ShapeDtypeStruct((16,), jnp.float32),
    mesh=plsc.VectorSubcoreMesh(core_axis_name="c", subcore_axis_name="s"),
    compiler_params=pltpu.CompilerParams(needs_layout_passes=False),
    scratch_types=[
        pltpu.VMEM((NR,), jnp.float32),
        pltpu.VMEM((NR,), jnp.int32),
        pltpu.VMEM((NR,), jnp.int32),
        pltpu.VMEM((NR,), jnp.int32),
        pltpu.VMEM((MATCH,), jnp.int32),
        pltpu.VMEM((16,), jnp.float32),
    ],
)
def _phase2(biou_hbm, bidx_hbm, rank_hbm, out_hbm,
            biou_v, bidx_v, rank_v, sortedg_v, matched_v, ap_v):
    @pl.when((lax.axis_index("c") == 0) & (lax.axis_index("s") == 0))
    def _():
        pltpu.sync_copy(biou_hbm, biou_v)
        pltpu.sync_copy(bidx_hbm, bidx_v)
        pltpu.sync_copy(rank_hbm, rank_v)

        zero16 = jnp.zeros((16,), jnp.int32)

        def zinit(c, _):
            matched_v[pl.ds(c * 16, 16)] = zero16
            return 0

        lax.fori_loop(0, MATCH // 16, zinit, 0)

        def loop_a(c, _):
            ds = pl.ds(c * 16, 16)
            gg = jnp.where(biou_v[ds] > IOU_THRESH, bidx_v[ds], SENT)
            plsc.store_scatter(sortedg_v, [rank_v[ds]], gg)
            return 0

        lax.fori_loop(0, NV, loop_a, 0)

        iota16 = lax.broadcasted_iota(jnp.int32, (16,), 0)
        ones_i = jnp.ones((16,), jnp.int32)

        def loop_b(c, carry):
            cum_carry, ap_acc = carry
            gg = sortedg_v[pl.ds(c * 16, 16)]
            m = plsc.load_gather(matched_v, [gg])
            _, lastm = plsc.scan_count(lax.rev(gg, (0,)))
            firsti = lax.rev(lastm.astype(jnp.int32), (0,))
            tpm = (gg < SENT) & (m == 0) & (firsti == 1)
            plsc.store_scatter(matched_v, [gg], ones_i, mask=tpm)
            tpf = jnp.where(tpm, 1.0, 0.0).astype(jnp.float32)
            cum = plsc.cumsum(tpf) + cum_carry
            new_carry = jnp.broadcast_to(jnp.max(cum), (16,))
            p = iota16 + c * 16
            pf = p.astype(jnp.float32)
            cum_prev = cum - tpf
            d = cum / DTOT - cum_prev / DTOT
            prec = cum / ((pf + 1.0) + 1e-6)
            prec_prev = jnp.where(p == 0, 1.0, cum_prev / (pf + 1e-6))
            ap_acc = ap_acc + d * (prec + prec_prev) * 0.5
            return (new_carry, ap_acc)

        _, ap_acc = lax.fori_loop(
            0, NV, loop_b,
            (jnp.zeros((16,), jnp.float32), jnp.zeros((16,), jnp.float32)))
        ap_v[...] = jnp.broadcast_to(jnp.sum(ap_acc), (16,))
        pltpu.sync_copy(ap_v, out_hbm)


@jax.jit
def kernel(pred_boxes, pred_scores, gt_boxes, pred_labels, gt_labels):
    del pred_labels, gt_labels
    prow = NR - N_PRED
    gcol = COLS - N_GT
    dx = jnp.pad(pred_boxes[:, 0], (0, prow))
    dy = jnp.pad(pred_boxes[:, 1], (0, prow))
    dx2 = dx + jnp.pad(pred_boxes[:, 2], (0, prow))
    dy2 = dy + jnp.pad(pred_boxes[:, 3], (0, prow))
    da = jnp.abs((dx2 - dx) * (dy2 - dy))
    dsc = jnp.pad(pred_scores, (0, prow), constant_values=-1.0)
    gx = jnp.pad(gt_boxes[:, 0], (0, gcol), constant_values=5.0)
    gy = jnp.pad(gt_boxes[:, 1], (0, gcol), constant_values=5.0)
    gx2 = gx + jnp.pad(gt_boxes[:, 2], (0, gcol))
    gy2 = gy + jnp.pad(gt_boxes[:, 3], (0, gcol))
    ga = jnp.abs((gx2 - gx) * (gy2 - gy))
    ssc = jnp.pad(pred_scores, (0, COLS - N_PRED),
                  constant_values=-1.0)

    biou, bidx, brank = _phase1(
        dx[:, None], dy[:, None], dx2[:, None], dy2[:, None],
        da[:, None], dsc[:, None],
        gx[None, :], gy[None, :], gx2[None, :], gy2[None, :],
        ga[None, :], ssc[None, :])
    out16 = _phase2(biou.reshape(NR), bidx.reshape(NR), brank.reshape(NR))
    return out16[0]


# speedup vs baseline: 1.2652x; 1.1234x over previous
import functools

import numpy as np

import jax
import jax.numpy as jnp
from jax import lax
from jax.experimental import pallas as pl
from jax.experimental.pallas import tpu as pltpu
from jax.experimental.pallas import tpu_sc as plsc

N_PRED = 20000
N_GT = 20000
IOU_THRESH = 0.5

ROWS_T = 256
NR = 20224
COLS = 20480
CT = 1024
NCH = COLS // CT
SENT = N_GT
MATCH = 20480
NV = NR // 16
DTOT = float(np.float32(np.float32(float(N_GT)) + np.float32(1e-6)))


def _phase1_body(dx, dy, dx2b, dy2b, da, dsc,
                 gx, gy, gx2b, gy2b, ga, ssc,
                 biou_ref, bidx_ref, brank_ref):
    i = pl.program_id(0)
    dxv = dx[:, :]
    dyv = dy[:, :]
    dx2 = dx2b[:, :]
    dy2 = dy2b[:, :]
    dav = da[:, :]
    dscv = dsc[:, :]
    rowid = i * ROWS_T + lax.broadcasted_iota(jnp.int32, (ROWS_T, 1), 0)
    colid0 = lax.broadcasted_iota(jnp.int32, (ROWS_T, CT), 1)

    rmax = jnp.full((ROWS_T, 1), -1.0, jnp.float32)
    ridx = jnp.zeros((ROWS_T, 1), jnp.int32)
    crank = jnp.zeros((ROWS_T, 1), jnp.int32)

    for c in range(NCH):
        sl = slice(c * CT, (c + 1) * CT)
        x1 = jnp.maximum(dxv, gx[:, sl])
        y1 = jnp.maximum(dyv, gy[:, sl])
        x2 = jnp.minimum(dx2, gx2b[:, sl])
        y2 = jnp.minimum(dy2, gy2b[:, sl])
        inter = jnp.maximum(x2 - x1, 0.0) * jnp.maximum(y2 - y1, 0.0)
        iou = inter / (((dav + ga[:, sl]) - inter) + 1e-6)
        cmax = jnp.max(iou, axis=1, keepdims=True)
        cidx = jnp.min(jnp.where(iou == cmax, colid0, CT),
                       axis=1, keepdims=True) + c * CT
        upd = cmax > rmax
        ridx = jnp.where(upd, cidx, ridx)
        rmax = jnp.where(upd, cmax, rmax)

        sv = ssc[:, sl]
        rmask = (sv > dscv) | ((sv == dscv) & (colid0 < (rowid - c * CT)))
        crank = crank + jnp.sum(rmask.astype(jnp.int32),
                                axis=1, keepdims=True)

    biou_ref[:, :] = rmax
    bidx_ref[:, :] = ridx
    brank_ref[:, :] = crank


_phase1 = pl.pallas_call(
    _phase1_body,
    grid=(NR // ROWS_T,),
    in_specs=(
        [pl.BlockSpec((ROWS_T, 1), lambda i: (i, 0)) for _ in range(6)]
        + [pl.BlockSpec((1, COLS), lambda i: (0, 0)) for _ in range(6)]
    ),
    out_specs=[pl.BlockSpec((ROWS_T, 1), lambda i: (i, 0)) for _ in range(3)],
    out_shape=[
        jax.ShapeDtypeStruct((NR, 1), jnp.float32),
        jax.ShapeDtypeStruct((NR, 1), jnp.int32),
        jax.ShapeDtypeStruct((NR, 1), jnp.int32),
    ],
)


@functools.partial(
    pl.kernel,
    out_type=jax.ShapeDtypeStruct((16,), jnp.float32),
    mesh=plsc.VectorSubcoreMesh(core_axis_name="c", subcore_axis_name="s"),
    compiler_params=pltpu.CompilerParams(needs_layout_passes=False),
    scratch_types=[
        pltpu.VMEM((NR,), jnp.float32),
        pltpu.VMEM((NR,), jnp.int32),
        pltpu.VMEM((NR,), jnp.int32),
        pltpu.VMEM((NR,), jnp.int32),
        pltpu.VMEM((MATCH,), jnp.int32),
        pltpu.VMEM((16,), jnp.float32),
    ],
)
def _phase2(biou_hbm, bidx_hbm, rank_hbm, out_hbm,
            biou_v, bidx_v, rank_v, sortedg_v, matched_v, ap_v):
    @pl.when((lax.axis_index("c") == 0) & (lax.axis_index("s") == 0))
    def _():
        pltpu.sync_copy(biou_hbm, biou_v)
        pltpu.sync_copy(bidx_hbm, bidx_v)
        pltpu.sync_copy(rank_hbm, rank_v)

        zero16 = jnp.zeros((16,), jnp.int32)

        def zinit(c, _):
            matched_v[pl.ds(c * 16, 16)] = zero16
            return 0

        lax.fori_loop(0, MATCH // 16, zinit, 0)

        def loop_a(c, _):
            ds = pl.ds(c * 16, 16)
            gg = jnp.where(biou_v[ds] > IOU_THRESH, bidx_v[ds], SENT)
            plsc.store_scatter(sortedg_v, [rank_v[ds]], gg)
            return 0

        lax.fori_loop(0, NV, loop_a, 0)

        iota16 = lax.broadcasted_iota(jnp.int32, (16,), 0)
        ones_i = jnp.ones((16,), jnp.int32)

        def loop_b(c, carry):
            cum_carry, ap_acc = carry
            gg = sortedg_v[pl.ds(c * 16, 16)]
            m = plsc.load_gather(matched_v, [gg])
            _, lastm = plsc.scan_count(lax.rev(gg, (0,)))
            firsti = lax.rev(lastm.astype(jnp.int32), (0,))
            tpm = (gg < SENT) & (m == 0) & (firsti == 1)
            plsc.store_scatter(matched_v, [gg], ones_i, mask=tpm)
            tpf = jnp.where(tpm, 1.0, 0.0).astype(jnp.float32)
            cum = plsc.cumsum(tpf) + cum_carry
            new_carry = jnp.broadcast_to(jnp.max(cum), (16,))
            p = iota16 + c * 16
            pf = p.astype(jnp.float32)
            cum_prev = cum - tpf
            d = cum / DTOT - cum_prev / DTOT
            prec = cum / ((pf + 1.0) + 1e-6)
            prec_prev = jnp.where(p == 0, 1.0, cum_prev / (pf + 1e-6))
            ap_acc = ap_acc + d * (prec + prec_prev) * 0.5
            return (new_carry, ap_acc)

        _, ap_acc = lax.fori_loop(
            0, NV, loop_b,
            (jnp.zeros((16,), jnp.float32), jnp.zeros((16,), jnp.float32)))
        ap_v[...] = jnp.broadcast_to(jnp.sum(ap_acc), (16,))
        pltpu.sync_copy(ap_v, out_hbm)


@jax.jit
def kernel(pred_boxes, pred_scores, gt_boxes, pred_labels, gt_labels):
    del pred_labels, gt_labels
    prow = NR - N_PRED
    gcol = COLS - N_GT
    dx = jnp.pad(pred_boxes[:, 0], (0, prow))
    dy = jnp.pad(pred_boxes[:, 1], (0, prow))
    dx2 = dx + jnp.pad(pred_boxes[:, 2], (0, prow))
    dy2 = dy + jnp.pad(pred_boxes[:, 3], (0, prow))
    da = jnp.abs((dx2 - dx) * (dy2 - dy))
    dsc = jnp.pad(pred_scores, (0, prow), constant_values=-1.0)
    gx = jnp.pad(gt_boxes[:, 0], (0, gcol), constant_values=5.0)
    gy = jnp.pad(gt_boxes[:, 1], (0, gcol), constant_values=5.0)
    gx2 = gx + jnp.pad(gt_boxes[:, 2], (0, gcol))
    gy2 = gy + jnp.pad(gt_boxes[:, 3], (0, gcol))
    ga = jnp.abs((gx2 - gx) * (gy2 - gy))
    ssc = jnp.pad(pred_scores, (0, COLS - N_PRED),
                  constant_values=-1.0)

    biou, bidx, brank = _phase1(
        dx[:, None], dy[:, None], dx2[:, None], dy2[:, None],
        da[:, None], dsc[:, None],
        gx[None, :], gy[None, :], gx2[None, :], gy2[None, :],
        ga[None, :], ssc[None, :])
    out16 = _phase2(biou.reshape(NR), bidx.reshape(NR), brank.reshape(NR))
    return out16[0]
